# baseline (speedup 1.0000x reference)
import jax
import jax.numpy as jnp
from jax import lax
from jax.experimental import pallas as pl
from jax.experimental.pallas import tpu as pltpu
from jax.experimental.pallas import tpu_sc as plsc

N = 24
E = 384
C = 512
NPAD = 32
LANES = 16
NTILES = 32
CH_PER_TILE = C // NTILES
BLK = 128


def _newton_rsqrt(d):
    i = plsc.bitcast(d, jnp.int32)
    y = plsc.bitcast(jnp.int32(0x5F3759DF) - (i >> 1), jnp.float32)
    for _ in range(3):
        y = y * (1.5 - 0.5 * d * y * y)
    return y


def _sc_tx1_body(ei_hbm, ew_hbm, x_hbm, out_hbm,
                 ei_v, ew_v, deg_v, dinv_v, wn_v, x_v, out_v, sem, xsem):
    wid = lax.axis_index("s") * 2 + lax.axis_index("c")
    ch0 = wid * CH_PER_TILE
    iota = lax.broadcasted_iota(jnp.int32, (LANES,), 0)
    ch_iota = ch0 + iota
    zeros = jnp.zeros((LANES,), jnp.float32)

    xcp = pltpu.async_copy(x_hbm, x_v, xsem)
    in_dmas = [
        pltpu.async_copy(ei_hbm, ei_v, sem),
        pltpu.async_copy(ew_hbm, ew_v, sem),
    ]
    for d in in_dmas:
        d.wait()

    for j in range(NPAD // LANES):
        deg_v[pl.ds(j * LANES, LANES)] = zeros
    for k in range(E // LANES):
        rv = ei_v[0, pl.ds(k * LANES, LANES)]
        wv = ew_v[pl.ds(k * LANES, LANES)]
        plsc.addupdate_scatter(deg_v, [rv], wv)

    for j in range(NPAD // LANES):
        d = deg_v[pl.ds(j * LANES, LANES)]
        dinv_v[pl.ds(j * LANES, LANES)] = jnp.where(
            d > 0.0, _newton_rsqrt(d), 0.0)

    for k in range(E // LANES):
        rv = ei_v[0, pl.ds(k * LANES, LANES)]
        cv = ei_v[1, pl.ds(k * LANES, LANES)]
        wv = ew_v[pl.ds(k * LANES, LANES)]
        dr = plsc.load_gather(dinv_v, [rv])
        dc = plsc.load_gather(dinv_v, [cv])
        wn_v[pl.ds(k * LANES, LANES)] = -(dr * wv * dc)

    for r in range(N):
        out_v[r, :] = zeros
    xcp.wait()

    xch = [jnp.full((LANES,), ch0 + c, jnp.int32) for c in range(CH_PER_TILE)]
    och = [jnp.full((LANES,), c, jnp.int32) for c in range(CH_PER_TILE)]

    @plsc.parallel_loop(0, E // LANES, 1, unroll=2)
    def _edge_chunks(k):
        base = k * LANES
        rv16 = ei_v[0, pl.ds(base, LANES)]
        cv16 = ei_v[1, pl.ds(base, LANES)]
        wn16 = wn_v[pl.ds(base, LANES)]
        for c in range(CH_PER_TILE):
            xg = plsc.load_gather(x_v, [cv16, xch[c]])
            plsc.addupdate_scatter(out_v, [rv16, och[c]], wn16 * xg)

    blk = ch0 // BLK
    j0 = ch0 % BLK
    out_dmas = [
        pltpu.async_copy(out_v.at[r],
                         out_hbm.at[pl.ds(blk * N * BLK + r * BLK + j0,
                                          CH_PER_TILE)], sem)
        for r in range(N)
    ]
    for d in out_dmas:
        d.wait()


@jax.jit
def _sc_tx1(ei, ew, x):
    mesh = plsc.VectorSubcoreMesh(core_axis_name="c", subcore_axis_name="s")
    f = pl.kernel(
        _sc_tx1_body, mesh=mesh,
        compiler_params=pltpu.CompilerParams(needs_layout_passes=False),
        out_type=jax.ShapeDtypeStruct((N * C,), jnp.float32),
        scratch_types=[
            pltpu.VMEM((2, E), jnp.int32),
            pltpu.VMEM((E,), jnp.float32),
            pltpu.VMEM((NPAD,), jnp.float32),
            pltpu.VMEM((NPAD,), jnp.float32),
            pltpu.VMEM((E,), jnp.float32),
            pltpu.VMEM((N, C), jnp.float32),
            pltpu.VMEM((N, CH_PER_TILE), jnp.float32),
            pltpu.SemaphoreType.DMA,
            pltpu.SemaphoreType.DMA,
        ],
    )
    return f(ei, ew, x)


def _tc_gates_kernel(x_ref, tx1_ref, wz_ref, wh_ref, uz_ref, uh_ref,
                     bz_ref, bh_ref, out_ref):
    f32 = jnp.float32
    x = x_ref[:]
    tx1 = jnp.concatenate([tx1_ref[b] for b in range(C // BLK)], axis=-1)
    cz = jax.nn.relu(
        jnp.dot(x, wz_ref[0], preferred_element_type=f32)
        + jnp.dot(tx1, wz_ref[1], preferred_element_type=f32))
    ch = jax.nn.relu(
        jnp.dot(x, wh_ref[0], preferred_element_type=f32)
        + jnp.dot(tx1, wh_ref[1], preferred_element_type=f32))
    z = jax.nn.sigmoid(
        jax.lax.dot_general(cz, uz_ref[:], (((1,), (1,)), ((), ())),
                            preferred_element_type=f32) + bz_ref[:])
    ht = jnp.tanh(
        jax.lax.dot_general(ch, uh_ref[:], (((1,), (1,)), ((), ())),
                            preferred_element_type=f32) + bh_ref[:])
    out_ref[:] = z * ht


def kernel(X, edge_index, edge_weight, W_xz, W_qz, W_xr, W_qr, W_xh, W_qh,
           w_x_z, w_q_z, w_x_r, w_q_r, w_x_h, w_q_h, b_z, b_r, b_h):
    ei = edge_index.astype(jnp.int32)
    ew = edge_weight.astype(jnp.float32)
    tx1_blocks = _sc_tx1(ei, ew, X).reshape(C // BLK, N, BLK)
    return pl.pallas_call(
        _tc_gates_kernel,
        out_shape=jax.ShapeDtypeStruct((N, C), jnp.float32),
    )(X, tx1_blocks, W_xz, W_xh, w_x_z, w_x_h, b_z, b_h)


# speedup vs baseline: 1.1955x; 1.1955x over previous
import jax
import jax.numpy as jnp
from jax import lax
from jax.experimental import pallas as pl
from jax.experimental.pallas import tpu as pltpu
from jax.experimental.pallas import tpu_sc as plsc

N = 24
E = 384
C = 512
NPAD = 32
LANES = 16
NTILES = 32
CH_PER_TILE = C // NTILES
BLK = 128


def _newton_rsqrt(d):
    i = plsc.bitcast(d, jnp.int32)
    y = plsc.bitcast(jnp.int32(0x5F3759DF) - (i >> 1), jnp.float32)
    for _ in range(3):
        y = y * (1.5 - 0.5 * d * y * y)
    return y


def _sc_tx1_body(ei_hbm, ew_hbm, x_hbm, out_hbm,
                 ei_v, ew_v, deg_v, dinv_v, wn_v, rc_v, x_v, out_v, sem, xsem):
    wid = lax.axis_index("s") * 2 + lax.axis_index("c")
    ch0 = wid * CH_PER_TILE
    iota = lax.broadcasted_iota(jnp.int32, (LANES,), 0)
    ch_iota = ch0 + iota
    zeros = jnp.zeros((LANES,), jnp.float32)

    xcp = pltpu.async_copy(x_hbm, x_v, xsem)
    in_dmas = [
        pltpu.async_copy(ei_hbm, ei_v, sem),
        pltpu.async_copy(ew_hbm, ew_v, sem),
    ]
    for d in in_dmas:
        d.wait()

    for j in range(NPAD // LANES):
        deg_v[pl.ds(j * LANES, LANES)] = zeros
    for k in range(E // LANES):
        rv = ei_v[0, pl.ds(k * LANES, LANES)]
        wv = ew_v[pl.ds(k * LANES, LANES)]
        plsc.addupdate_scatter(deg_v, [rv], wv)

    for j in range(NPAD // LANES):
        d = deg_v[pl.ds(j * LANES, LANES)]
        dinv_v[pl.ds(j * LANES, LANES)] = jnp.where(
            d > 0.0, _newton_rsqrt(d), 0.0)

    for k in range(E // LANES):
        rv = ei_v[0, pl.ds(k * LANES, LANES)]
        cv = ei_v[1, pl.ds(k * LANES, LANES)]
        wv = ew_v[pl.ds(k * LANES, LANES)]
        dr = plsc.load_gather(dinv_v, [rv])
        dc = plsc.load_gather(dinv_v, [cv])
        wn_v[pl.ds(k * LANES, LANES)] = -(dr * wv * dc)
        rc_v[pl.ds(k * LANES, LANES)] = (rv << 5) | cv

    for r in range(N):
        out_v[r, :] = zeros
    xcp.wait()

    @plsc.parallel_loop(0, E // LANES, 1, unroll=4)
    def _edge_chunks(k):
        base = k * LANES
        for l in range(LANES):
            ev = jnp.full((LANES,), base + l, jnp.int32)
            rc = plsc.load_gather(rc_v, [ev])
            wb = plsc.load_gather(wn_v, [ev])
            rb = rc >> 5
            cb = rc & 31
            xr = plsc.load_gather(x_v, [cb, ch_iota])
            plsc.addupdate_scatter(out_v, [rb, iota], wb * xr)

    blk = ch0 // BLK
    j0 = ch0 % BLK
    out_dmas = [
        pltpu.async_copy(out_v.at[r],
                         out_hbm.at[pl.ds(blk * N * BLK + r * BLK + j0,
                                          CH_PER_TILE)], sem)
        for r in range(N)
    ]
    for d in out_dmas:
        d.wait()


@jax.jit
def _sc_tx1(ei, ew, x):
    mesh = plsc.VectorSubcoreMesh(core_axis_name="c", subcore_axis_name="s")
    f = pl.kernel(
        _sc_tx1_body, mesh=mesh,
        compiler_params=pltpu.CompilerParams(needs_layout_passes=False),
        out_type=jax.ShapeDtypeStruct((N * C,), jnp.float32),
        scratch_types=[
            pltpu.VMEM((2, E), jnp.int32),
            pltpu.VMEM((E,), jnp.float32),
            pltpu.VMEM((NPAD,), jnp.float32),
            pltpu.VMEM((NPAD,), jnp.float32),
            pltpu.VMEM((E,), jnp.float32),
            pltpu.VMEM((E,), jnp.int32),
            pltpu.VMEM((N, C), jnp.float32),
            pltpu.VMEM((N, CH_PER_TILE), jnp.float32),
            pltpu.SemaphoreType.DMA,
            pltpu.SemaphoreType.DMA,
        ],
    )
    return f(ei, ew, x)


def _tc_x0_kernel(x_ref, wz_ref, wh_ref, xz_ref, xh_ref):
    f32 = jnp.float32
    x = x_ref[:]
    xz_ref[:] = jnp.dot(x, wz_ref[0], preferred_element_type=f32)
    xh_ref[:] = jnp.dot(x, wh_ref[0], preferred_element_type=f32)


def _tc_gates_kernel(xz_ref, xh_ref, tx1_ref, wz_ref, wh_ref, uz_ref,
                     uh_ref, bz_ref, bh_ref, out_ref):
    f32 = jnp.float32
    tx1 = jnp.concatenate([tx1_ref[b] for b in range(C // BLK)], axis=-1)
    cz = jax.nn.relu(
        xz_ref[:] + jnp.dot(tx1, wz_ref[1], preferred_element_type=f32))
    ch = jax.nn.relu(
        xh_ref[:] + jnp.dot(tx1, wh_ref[1], preferred_element_type=f32))
    z = jax.nn.sigmoid(
        jax.lax.dot_general(cz, uz_ref[:], (((1,), (1,)), ((), ())),
                            preferred_element_type=f32) + bz_ref[:])
    ht = jnp.tanh(
        jax.lax.dot_general(ch, uh_ref[:], (((1,), (1,)), ((), ())),
                            preferred_element_type=f32) + bh_ref[:])
    out_ref[:] = z * ht


def kernel(X, edge_index, edge_weight, W_xz, W_qz, W_xr, W_qr, W_xh, W_qh,
           w_x_z, w_q_z, w_x_r, w_q_r, w_x_h, w_q_h, b_z, b_r, b_h):
    ei = edge_index.astype(jnp.int32)
    ew = edge_weight.astype(jnp.float32)
    tx1_blocks = _sc_tx1(ei, ew, X).reshape(C // BLK, N, BLK)
    xz0, xh0 = pl.pallas_call(
        _tc_x0_kernel,
        out_shape=[jax.ShapeDtypeStruct((N, C), jnp.float32),
                   jax.ShapeDtypeStruct((N, C), jnp.float32)],
    )(X, W_xz, W_xh)
    return pl.pallas_call(
        _tc_gates_kernel,
        out_shape=jax.ShapeDtypeStruct((N, C), jnp.float32),
    )(xz0, xh0, tx1_blocks, W_xz, W_xh, w_x_z, w_x_h, b_z, b_h)


# speedup vs baseline: 1.4651x; 1.2255x over previous
import jax
import jax.numpy as jnp
from jax import lax
from jax.experimental import pallas as pl
from jax.experimental.pallas import tpu as pltpu
from jax.experimental.pallas import tpu_sc as plsc

N = 24
E = 384
C = 512
NPAD = 32
LANES = 16


def _newton_rsqrt(d):
    i = plsc.bitcast(d, jnp.int32)
    y = plsc.bitcast(jnp.int32(0x5F3759DF) - (i >> 1), jnp.float32)
    for _ in range(3):
        y = y * (1.5 - 0.5 * d * y * y)
    return y


def _sc_adj_body(ei_hbm, ew_hbm, a_hbm, ei_v, ew_v, deg_v, dinv_v, a_v, sem):
    wid = lax.axis_index("s") * 2 + lax.axis_index("c")

    @pl.when(wid == 0)
    def _():
        zeros = jnp.zeros((LANES,), jnp.float32)

        in_dmas = [
            pltpu.async_copy(ei_hbm, ei_v, sem),
            pltpu.async_copy(ew_hbm, ew_v, sem),
        ]
        for r in range(N):
            a_v[r, pl.ds(0, LANES)] = zeros
            a_v[r, pl.ds(LANES, LANES)] = zeros
        for d in in_dmas:
            d.wait()

        for j in range(NPAD // LANES):
            deg_v[pl.ds(j * LANES, LANES)] = zeros
        for k in range(E // LANES):
            rv = ei_v[0, pl.ds(k * LANES, LANES)]
            wv = ew_v[pl.ds(k * LANES, LANES)]
            plsc.addupdate_scatter(deg_v, [rv], wv)

        for j in range(NPAD // LANES):
            d = deg_v[pl.ds(j * LANES, LANES)]
            dinv_v[pl.ds(j * LANES, LANES)] = jnp.where(
                d > 0.0, _newton_rsqrt(d), 0.0)

        for k in range(E // LANES):
            rv = ei_v[0, pl.ds(k * LANES, LANES)]
            cv = ei_v[1, pl.ds(k * LANES, LANES)]
            wv = ew_v[pl.ds(k * LANES, LANES)]
            dr = plsc.load_gather(dinv_v, [rv])
            dc = plsc.load_gather(dinv_v, [cv])
            plsc.addupdate_scatter(a_v, [rv, cv], -(dr * wv * dc))

        pltpu.sync_copy(a_v, a_hbm)


@jax.jit
def _sc_adj(ei, ew):
    mesh = plsc.VectorSubcoreMesh(core_axis_name="c", subcore_axis_name="s")
    f = pl.kernel(
        _sc_adj_body, mesh=mesh,
        compiler_params=pltpu.CompilerParams(needs_layout_passes=False),
        out_type=jax.ShapeDtypeStruct((N, NPAD), jnp.float32),
        scratch_types=[
            pltpu.VMEM((2, E), jnp.int32),
            pltpu.VMEM((E,), jnp.float32),
            pltpu.VMEM((NPAD,), jnp.float32),
            pltpu.VMEM((NPAD,), jnp.float32),
            pltpu.VMEM((N, NPAD), jnp.float32),
            pltpu.SemaphoreType.DMA,
        ],
    )
    return f(ei, ew)


def _tc_xmm_kernel(x_ref, wz_ref, wh_ref, xz0_ref, xz1_ref, xh0_ref, xh1_ref):
    f32 = jnp.float32
    x = x_ref[:]
    xz0_ref[:] = jnp.dot(x, wz_ref[0], preferred_element_type=f32)
    xz1_ref[:] = jnp.dot(x, wz_ref[1], preferred_element_type=f32)
    xh0_ref[:] = jnp.dot(x, wh_ref[0], preferred_element_type=f32)
    xh1_ref[:] = jnp.dot(x, wh_ref[1], preferred_element_type=f32)


def _tc_gates_kernel(a_ref, xz0_ref, xz1_ref, xh0_ref, xh1_ref,
                     uz_ref, uh_ref, bz_ref, bh_ref, out_ref):
    f32 = jnp.float32
    a = a_ref[:, :N]
    cz = jax.nn.relu(
        xz0_ref[:] + jnp.dot(a, xz1_ref[:], preferred_element_type=f32))
    ch = jax.nn.relu(
        xh0_ref[:] + jnp.dot(a, xh1_ref[:], preferred_element_type=f32))
    z = jax.nn.sigmoid(
        jax.lax.dot_general(cz, uz_ref[:], (((1,), (1,)), ((), ())),
                            preferred_element_type=f32) + bz_ref[:])
    ht = jnp.tanh(
        jax.lax.dot_general(ch, uh_ref[:], (((1,), (1,)), ((), ())),
                            preferred_element_type=f32) + bh_ref[:])
    out_ref[:] = z * ht


def kernel(X, edge_index, edge_weight, W_xz, W_qz, W_xr, W_qr, W_xh, W_qh,
           w_x_z, w_q_z, w_x_r, w_q_r, w_x_h, w_q_h, b_z, b_r, b_h):
    ei = edge_index.astype(jnp.int32)
    ew = edge_weight.astype(jnp.float32)
    a = _sc_adj(ei, ew)
    xz0, xz1, xh0, xh1 = pl.pallas_call(
        _tc_xmm_kernel,
        out_shape=[jax.ShapeDtypeStruct((N, C), jnp.float32)] * 4,
    )(X, W_xz, W_xh)
    return pl.pallas_call(
        _tc_gates_kernel,
        out_shape=jax.ShapeDtypeStruct((N, C), jnp.float32),
    )(a, xz0, xz1, xh0, xh1, w_x_z, w_x_h, b_z, b_h)


# speedup vs baseline: 1.5717x; 1.0728x over previous
import jax
import jax.numpy as jnp
from jax import lax
from jax.experimental import pallas as pl
from jax.experimental.pallas import tpu as pltpu
from jax.experimental.pallas import tpu_sc as plsc

N = 24
E = 384
C = 512
NPAD = 32
LANES = 16


def _newton_rsqrt(d):
    i = plsc.bitcast(d, jnp.int32)
    y = plsc.bitcast(jnp.int32(0x5F3759DF) - (i >> 1), jnp.float32)
    for _ in range(3):
        y = y * (1.5 - 0.5 * d * y * y)
    return y


def _sc_adj_body(ei_hbm, ew_hbm, a_hbm, ei_v, ew_v, deg_v, dinv_v, a_v, sem):
    wid = lax.axis_index("s") * 2 + lax.axis_index("c")

    @pl.when(wid == 0)
    def _():
        zeros = jnp.zeros((LANES,), jnp.float32)

        in_dmas = [
            pltpu.async_copy(ei_hbm, ei_v, sem),
            pltpu.async_copy(ew_hbm, ew_v, sem),
        ]
        for r in range(N):
            a_v[r, pl.ds(0, LANES)] = zeros
            a_v[r, pl.ds(LANES, LANES)] = zeros
        for d in in_dmas:
            d.wait()

        for j in range(NPAD // LANES):
            deg_v[pl.ds(j * LANES, LANES)] = zeros
        for k in range(E // LANES):
            rv = ei_v[0, pl.ds(k * LANES, LANES)]
            wv = ew_v[pl.ds(k * LANES, LANES)]
            plsc.addupdate_scatter(deg_v, [rv], wv)

        for j in range(NPAD // LANES):
            d = deg_v[pl.ds(j * LANES, LANES)]
            dinv_v[pl.ds(j * LANES, LANES)] = jnp.where(
                d > 0.0, _newton_rsqrt(d), 0.0)

        for k in range(E // LANES):
            rv = ei_v[0, pl.ds(k * LANES, LANES)]
            cv = ei_v[1, pl.ds(k * LANES, LANES)]
            wv = ew_v[pl.ds(k * LANES, LANES)]
            dr = plsc.load_gather(dinv_v, [rv])
            dc = plsc.load_gather(dinv_v, [cv])
            plsc.addupdate_scatter(a_v, [rv, cv], -(dr * wv * dc))

        pltpu.sync_copy(a_v, a_hbm)


@jax.jit
def _sc_adj(ei, ew):
    mesh = plsc.VectorSubcoreMesh(core_axis_name="c", subcore_axis_name="s",
                                  num_cores=1)
    f = pl.kernel(
        _sc_adj_body, mesh=mesh,
        compiler_params=pltpu.CompilerParams(needs_layout_passes=False),
        out_type=jax.ShapeDtypeStruct((N, NPAD), jnp.float32),
        scratch_types=[
            pltpu.VMEM((2, E), jnp.int32),
            pltpu.VMEM((E,), jnp.float32),
            pltpu.VMEM((NPAD,), jnp.float32),
            pltpu.VMEM((NPAD,), jnp.float32),
            pltpu.VMEM((N, NPAD), jnp.float32),
            pltpu.SemaphoreType.DMA,
        ],
    )
    return f(ei, ew)


def _tc_xmm_kernel(x_ref, wz_ref, wh_ref, xz0_ref, xz1_ref, xh0_ref, xh1_ref):
    f32 = jnp.float32
    x = x_ref[:]
    xz0_ref[:] = jnp.dot(x, wz_ref[0], preferred_element_type=f32)
    xz1_ref[:] = jnp.dot(x, wz_ref[1], preferred_element_type=f32)
    xh0_ref[:] = jnp.dot(x, wh_ref[0], preferred_element_type=f32)
    xh1_ref[:] = jnp.dot(x, wh_ref[1], preferred_element_type=f32)


def _tc_gates_kernel(a_ref, xz0_ref, xz1_ref, xh0_ref, xh1_ref,
                     uz_ref, uh_ref, bz_ref, bh_ref, out_ref):
    f32 = jnp.float32
    a = a_ref[:, :N]
    cz = jax.nn.relu(
        xz0_ref[:] + jnp.dot(a, xz1_ref[:], preferred_element_type=f32))
    ch = jax.nn.relu(
        xh0_ref[:] + jnp.dot(a, xh1_ref[:], preferred_element_type=f32))
    z = jax.nn.sigmoid(
        jax.lax.dot_general(cz, uz_ref[:], (((1,), (1,)), ((), ())),
                            preferred_element_type=f32) + bz_ref[:])
    ht = jnp.tanh(
        jax.lax.dot_general(ch, uh_ref[:], (((1,), (1,)), ((), ())),
                            preferred_element_type=f32) + bh_ref[:])
    out_ref[:] = z * ht


def kernel(X, edge_index, edge_weight, W_xz, W_qz, W_xr, W_qr, W_xh, W_qh,
           w_x_z, w_q_z, w_x_r, w_q_r, w_x_h, w_q_h, b_z, b_r, b_h):
    ei = edge_index.astype(jnp.int32)
    ew = edge_weight.astype(jnp.float32)
    a = _sc_adj(ei, ew)
    xz0, xz1, xh0, xh1 = pl.pallas_call(
        _tc_xmm_kernel,
        out_shape=[jax.ShapeDtypeStruct((N, C), jnp.float32)] * 4,
    )(X, W_xz, W_xh)
    return pl.pallas_call(
        _tc_gates_kernel,
        out_shape=jax.ShapeDtypeStruct((N, C), jnp.float32),
    )(a, xz0, xz1, xh0, xh1, w_x_z, w_x_h, b_z, b_h)
